# baseline (speedup 1.0000x reference)
import functools

import jax
import jax.numpy as jnp
from jax import lax
from jax.experimental import pallas as pl
from jax.experimental.pallas import tpu as pltpu
from jax.experimental.pallas import tpu_sc as plsc

_LOG2 = 0.6931471805599453
_L = 16
_NW = 32

_C6 = -0.017029610589110285
_C5 = 0.08152317761753866
_C4 = -0.18901954822312647
_C3 = 0.3150412799087584
_C2 = -0.4972033312202197
_C1 = 0.9998325947816336
_C0 = 1.6936626599750422e-06


def _body(n, out_hbm, t_hbm, ei_hbm, part_hbm, out_v, t_v, ei_v, res_v):
    chunks = n // _L
    rows_per_w = n // _NW
    cid = lax.axis_index("c")
    sid = lax.axis_index("s")
    wid = sid * 2 + cid

    pltpu.sync_copy(out_hbm, out_v.at[0:n])
    pltpu.sync_copy(t_hbm, t_v.at[0:n])
    pltpu.sync_copy(ei_hbm, ei_v.at[0:n])

    iota_f = lax.iota(jnp.int32, _L).astype(jnp.float32)
    zeros = jnp.zeros((_L,), jnp.float32)
    res_v[0:_L] = zeros
    res_v[_L:2 * _L] = zeros

    def row_body(r, carry):
        row = wid + _NW * r
        ei_a = ei_v[pl.ds(row, _L)][0]

        @pl.when(ei_a != 0.0)
        def do_row():
            out_a = jnp.full((_L,), out_v[pl.ds(row, _L)][0], jnp.float32)
            t_a = jnp.full((_L,), t_v[pl.ds(row, _L)][0], jnp.float32)
            row_f = jnp.full((_L,), row.astype(jnp.float32), jnp.float32)

            def chunk_body(k, cc):
                racc, rcnt, colf = cc
                col = k * _L
                ob = out_v[pl.ds(col, _L)]
                tb = t_v[pl.ds(col, _L)]
                bidx = iota_f + colf
                mask = (t_a < tb) | ((t_a == tb) & (row_f < bidx))
                v = out_a - ob
                e = jnp.exp(-jnp.abs(v))
                p = _C6
                p = p * e + _C5
                p = p * e + _C4
                p = p * e + _C3
                p = p * e + _C2
                p = p * e + _C1
                p = p * e + _C0
                ls = jnp.minimum(v, 0.0) - p
                racc = racc + jnp.where(mask, ls, 0.0)
                rcnt = rcnt + jnp.where(mask, 1.0, 0.0)
                return racc, rcnt, colf + 16.0

            racc, rcnt, _ = lax.fori_loop(0, chunks, chunk_body,
                                          (zeros, zeros, zeros))
            res_v[0:_L] = res_v[0:_L] + racc
            res_v[_L:2 * _L] = res_v[_L:2 * _L] + rcnt

        return carry

    lax.fori_loop(0, rows_per_w, row_body, 0)
    pltpu.sync_copy(res_v, part_hbm.at[wid])


def kernel(output, event_time, event_indicator):
    n = output.shape[0]
    ei_f = event_indicator.astype(jnp.float32)
    mesh = plsc.VectorSubcoreMesh(core_axis_name="c", subcore_axis_name="s")
    k = pl.kernel(
        functools.partial(_body, n),
        out_type=jax.ShapeDtypeStruct((_NW, 2 * _L), jnp.float32),
        mesh=mesh,
        scratch_types=[
            pltpu.VMEM((n + _L,), jnp.float32),
            pltpu.VMEM((n + _L,), jnp.float32),
            pltpu.VMEM((n + _L,), jnp.float32),
            pltpu.VMEM((2 * _L,), jnp.float32),
        ],
    )
    part = k(output, event_time, ei_f)
    s = jnp.sum(part[:, :_L])
    cnt = jnp.sum(part[:, _L:])
    return -(cnt + s / _LOG2) / cnt
